# baseline (speedup 1.0000x reference)
import functools

import jax
import jax.numpy as jnp
from jax.experimental import pallas as pl
from jax.experimental.pallas import tpu as pltpu


def _fused_heads_kernel(x_ref, w_ref, b_ref, coords_ref, pres_ref):
    y = jnp.dot(x_ref[...], w_ref[...], preferred_element_type=jnp.float32)
    y = y + b_ref[...]
    coords_ref[...] = y[:, 0:4]
    pres_ref[...] = y[:, 4:5]


@functools.partial(jax.jit, static_argnames=("tile",))
def _run(x, w, b, tile):
    n = x.shape[0]
    grid = (n // tile,)
    coords, pres = pl.pallas_call(
        _fused_heads_kernel,
        grid=grid,
        in_specs=[
            pl.BlockSpec((tile, x.shape[1]), lambda i: (i, 0)),
            pl.BlockSpec(w.shape, lambda i: (0, 0)),
            pl.BlockSpec(b.shape, lambda i: (0, 0)),
        ],
        out_specs=[
            pl.BlockSpec((tile, 4), lambda i: (i, 0)),
            pl.BlockSpec((tile, 1), lambda i: (i, 0)),
        ],
        out_shape=[
            jax.ShapeDtypeStruct((n, 4), jnp.float32),
            jax.ShapeDtypeStruct((n, 1), jnp.float32),
        ],
        compiler_params=pltpu.CompilerParams(
            dimension_semantics=("arbitrary",),
        ),
    )(x, w, b)
    return coords, pres


def kernel(local_features, W_coords, b_coords, W_pres, b_pres):
    B, C, R, D = local_features.shape
    n = B * C * R
    x = local_features.reshape(n, D)
    w = jnp.concatenate(
        [W_coords, W_pres, jnp.zeros((3, D), jnp.float32)], axis=0
    ).T
    b = jnp.concatenate(
        [b_coords, b_pres, jnp.zeros((3,), jnp.float32)]
    ).reshape(1, 8)
    coords, pres = _run(x, w, b, 3000)
    return (
        coords.reshape(B, C, R, 4),
        pres.reshape(B, C, R, 1),
    )
